# Initial kernel scaffold; baseline (speedup 1.0000x reference)
import jax
import jax.numpy as jnp
from jax.experimental import pallas as pl


def kernel(x, edge_index, W1, b1, W2, b2):
    raise NotImplementedError("write your pallas kernel here")


# speedup vs baseline: 13.5226x; 13.5226x over previous
import functools

import jax
import jax.numpy as jnp
from jax import lax
from jax.experimental import pallas as pl
from jax.experimental.pallas import tpu as pltpu
from jax.experimental.pallas import tpu_sc as plsc

N = 10000
D = 128
E = 320000

NC = 2
NS = 16
NW = NC * NS
EPW = E // NW
CH = 80
NCH = EPW // CH
NP = 10240
RPS = NP // NS
ZR = 128
DEGW = 16

_MESH = plsc.VectorSubcoreMesh(
    core_axis_name="c", subcore_axis_name="s", num_cores=NC, num_subcores=NS
)


@functools.partial(
    pl.kernel,
    out_type=jax.ShapeDtypeStruct((NC, NP, DEGW), jnp.float32),
    mesh=_MESH,
    scratch_types=[
        pltpu.VMEM_SHARED((NP, DEGW), jnp.float32),
        pltpu.VMEM((CH, DEGW), jnp.float32),
        pltpu.VMEM((CH,), jnp.int32),
        pltpu.VMEM((RPS, DEGW), jnp.float32),
    ],
)
def _sc_deg(dst_hbm, degp_hbm, acc, ones_v, idx_d, zbuf):
    c = lax.axis_index("c")
    s = lax.axis_index("s")
    w = c * NS + s

    zeros16 = jnp.zeros((16,), jnp.float32)
    ones16 = jnp.ones((16,), jnp.float32)

    def _fill(i, _):
        zbuf[i] = zeros16
        return _

    lax.fori_loop(0, RPS, _fill, 0)

    def _fill1(i, _):
        ones_v[i] = ones16
        return _

    lax.fori_loop(0, CH, _fill1, 0)
    pltpu.sync_copy(zbuf, acc.at[pl.ds(s * RPS, RPS)])
    plsc.subcore_barrier()

    base = w * EPW

    def _chunk(i, _):
        pltpu.sync_copy(dst_hbm.at[pl.ds(base + i * CH, CH)], idx_d)
        pltpu.sync_copy(ones_v, acc.at[idx_d], add=True)
        return _

    lax.fori_loop(0, NCH, _chunk, 0)
    plsc.subcore_barrier()
    pltpu.sync_copy(acc.at[pl.ds(s * RPS, RPS)], degp_hbm.at[c, pl.ds(s * RPS, RPS)])


@functools.partial(
    pl.kernel,
    out_type=jax.ShapeDtypeStruct((NC, NP, D), jnp.float32),
    mesh=_MESH,
    scratch_types=[
        pltpu.VMEM_SHARED((NP, D), jnp.float32),
        pltpu.VMEM((CH,), jnp.int32),
        pltpu.VMEM((CH,), jnp.int32),
        pltpu.VMEM((CH, D), jnp.float32),
        pltpu.VMEM((ZR, D), jnp.float32),
        pltpu.SemaphoreType.DMA,
    ],
)
def _sc_agg(g_hbm, src_hbm, dst_hbm, out_hbm, acc, idx_s, idx_d, rows, zbuf, sem):
    c = lax.axis_index("c")
    s = lax.axis_index("s")
    w = c * NS + s

    zeros16 = jnp.zeros((16,), jnp.float32)

    def _fill(i, _):
        for j in range(D // 16):
            zbuf[i, pl.ds(j * 16, 16)] = zeros16
        return _

    lax.fori_loop(0, ZR, _fill, 0)

    def _zero(k, _):
        pltpu.sync_copy(zbuf, acc.at[pl.ds(s * RPS + k * ZR, ZR)])
        return _

    lax.fori_loop(0, RPS // ZR, _zero, 0)
    plsc.subcore_barrier()

    base = w * EPW

    def _chunk(i, _):
        off = base + i * CH
        pltpu.sync_copy(src_hbm.at[pl.ds(off, CH)], idx_s)
        pltpu.sync_copy(dst_hbm.at[pl.ds(off, CH)], idx_d)
        pltpu.async_copy(g_hbm.at[idx_s], rows, sem).wait()
        pltpu.sync_copy(rows, acc.at[idx_d], add=True)
        return _

    lax.fori_loop(0, NCH, _chunk, 0)
    plsc.subcore_barrier()
    pltpu.sync_copy(acc.at[pl.ds(s * RPS, RPS)], out_hbm.at[c, pl.ds(s * RPS, RPS)])


BR = 2000


def _tc1_body(degp_ref, x_ref, w_ref, dis_ref, g_ref):
    deg = degp_ref[0, :, 0:1] + degp_ref[1, :, 0:1] + 1.0
    dis = lax.rsqrt(deg)
    dis_ref[...] = jnp.broadcast_to(dis, (BR, DEGW))
    g_ref[...] = jnp.dot(x_ref[...], w_ref[...], preferred_element_type=jnp.float32) * dis


def _tc2_body(sp_ref, g_ref, dis_ref, w_ref, b_ref, g2_ref):
    dis = dis_ref[:, 0:1]
    h = (sp_ref[0] + sp_ref[1] + g_ref[...]) * dis + b_ref[...]
    h = jnp.maximum(h, 0.0)
    g2_ref[...] = jnp.dot(h, w_ref[...], preferred_element_type=jnp.float32) * dis


def _tc3_body(sp_ref, g_ref, dis_ref, b_ref, out_ref):
    dis = dis_ref[:, 0:1]
    h = (sp_ref[0] + sp_ref[1] + g_ref[...]) * dis + b_ref[...]
    m = jnp.max(h, axis=1, keepdims=True)
    ex = jnp.exp(h - m)
    out_ref[...] = h - m - jnp.log(jnp.sum(ex, axis=1, keepdims=True))


_tc1 = pl.pallas_call(
    _tc1_body,
    grid=(N // BR,),
    in_specs=[
        pl.BlockSpec((NC, BR, DEGW), lambda i: (0, i, 0)),
        pl.BlockSpec((BR, D), lambda i: (i, 0)),
        pl.BlockSpec((D, D), lambda i: (0, 0)),
    ],
    out_specs=[
        pl.BlockSpec((BR, DEGW), lambda i: (i, 0)),
        pl.BlockSpec((BR, D), lambda i: (i, 0)),
    ],
    out_shape=[
        jax.ShapeDtypeStruct((N, DEGW), jnp.float32),
        jax.ShapeDtypeStruct((N, D), jnp.float32),
    ],
)

_tc2 = pl.pallas_call(
    _tc2_body,
    grid=(N // BR,),
    in_specs=[
        pl.BlockSpec((NC, BR, D), lambda i: (0, i, 0)),
        pl.BlockSpec((BR, D), lambda i: (i, 0)),
        pl.BlockSpec((BR, DEGW), lambda i: (i, 0)),
        pl.BlockSpec((D, D), lambda i: (0, 0)),
        pl.BlockSpec((1, D), lambda i: (0, 0)),
    ],
    out_specs=pl.BlockSpec((BR, D), lambda i: (i, 0)),
    out_shape=jax.ShapeDtypeStruct((N, D), jnp.float32),
)

_tc3 = pl.pallas_call(
    _tc3_body,
    grid=(N // BR,),
    in_specs=[
        pl.BlockSpec((NC, BR, D), lambda i: (0, i, 0)),
        pl.BlockSpec((BR, D), lambda i: (i, 0)),
        pl.BlockSpec((BR, DEGW), lambda i: (i, 0)),
        pl.BlockSpec((1, D), lambda i: (0, 0)),
    ],
    out_specs=pl.BlockSpec((BR, D), lambda i: (i, 0)),
    out_shape=jax.ShapeDtypeStruct((N, D), jnp.float32),
)


@jax.jit
def kernel(x, edge_index, W1, b1, W2, b2):
    src = edge_index[0]
    dst = edge_index[1]
    degp = _sc_deg(dst)
    dis16, g1 = _tc1(degp, x, W1)
    s1 = _sc_agg(g1, src, dst)
    g2 = _tc2(s1, g1, dis16, W2, b1.reshape(1, D))
    s2 = _sc_agg(g2, src, dst)
    return _tc3(s2, g2, dis16, b2.reshape(1, D))


# speedup vs baseline: 25.6164x; 1.8943x over previous
import functools

import jax
import jax.numpy as jnp
from jax import lax
from jax.experimental import pallas as pl
from jax.experimental.pallas import tpu as pltpu
from jax.experimental.pallas import tpu_sc as plsc

N = 10000
D = 128
E = 320000

NC = 2
NS = 16
NW = NC * NS
EPW = E // NW
CH = 80
NCH = EPW // CH
NP = 10240
RPS = NP // NS
ZR = 128
DEGW = 16
NB = 5
NG = NCH // NB

_MESH = plsc.VectorSubcoreMesh(
    core_axis_name="c", subcore_axis_name="s", num_cores=NC, num_subcores=NS
)


@functools.partial(
    pl.kernel,
    out_type=jax.ShapeDtypeStruct((NC, NP, DEGW), jnp.float32),
    mesh=_MESH,
    scratch_types=[
        pltpu.VMEM_SHARED((NP, DEGW), jnp.float32),
        pltpu.VMEM((CH, DEGW), jnp.float32),
        pltpu.VMEM((CH,), jnp.int32),
        pltpu.VMEM((RPS, DEGW), jnp.float32),
    ],
)
def _sc_deg(dst_hbm, degp_hbm, acc, ones_v, idx_d, zbuf):
    c = lax.axis_index("c")
    s = lax.axis_index("s")
    w = c * NS + s

    zeros16 = jnp.zeros((16,), jnp.float32)
    ones16 = jnp.ones((16,), jnp.float32)

    def _fill(i, _):
        zbuf[i] = zeros16
        return _

    lax.fori_loop(0, RPS, _fill, 0)

    def _fill1(i, _):
        ones_v[i] = ones16
        return _

    lax.fori_loop(0, CH, _fill1, 0)
    pltpu.sync_copy(zbuf, acc.at[pl.ds(s * RPS, RPS)])
    plsc.subcore_barrier()

    base = w * EPW

    def _chunk(i, _):
        pltpu.sync_copy(dst_hbm.at[pl.ds(base + i * CH, CH)], idx_d)
        pltpu.sync_copy(ones_v, acc.at[idx_d], add=True)
        return _

    lax.fori_loop(0, NCH, _chunk, 0)
    plsc.subcore_barrier()
    pltpu.sync_copy(acc.at[pl.ds(s * RPS, RPS)], degp_hbm.at[c, pl.ds(s * RPS, RPS)])


NBUF = 4
BCH = 25
NBLK = NCH // BCH


@functools.partial(
    pl.kernel,
    out_type=jax.ShapeDtypeStruct((NC, NP, D), jnp.float32),
    mesh=_MESH,
    scratch_types=[
        pltpu.VMEM_SHARED((NP, D), jnp.float32),
        pltpu.VMEM((BCH, CH), jnp.int32),
        pltpu.VMEM((BCH, CH), jnp.int32),
        pltpu.VMEM((NBUF, CH, D), jnp.float32),
        [pltpu.SemaphoreType.DMA] * NBUF,
        [pltpu.SemaphoreType.DMA] * NBUF,
    ],
)
def _sc_agg(g_hbm, src_hbm, dst_hbm, out_hbm, acc, idx_s, idx_d, rows, gsems, ssems):
    c = lax.axis_index("c")
    s = lax.axis_index("s")
    w = c * NS + s

    zeros16 = jnp.zeros((16,), jnp.float32)

    def _fill(i, _):
        for b in range(NBUF):
            for j in range(D // 16):
                rows[b, i, pl.ds(j * 16, 16)] = zeros16
        return _

    lax.fori_loop(0, CH, _fill, 0)

    def _zero(k, _):
        for b in range(NBUF):
            pltpu.sync_copy(
                rows.at[b], acc.at[pl.ds(s * RPS + (k * NBUF + b) * CH, CH)]
            )
        return _

    lax.fori_loop(0, RPS // (NBUF * CH), _zero, 0)
    plsc.subcore_barrier()

    def _block(blk, _):
        pltpu.sync_copy(src_hbm.at[w * NBLK + blk], idx_s)
        pltpu.sync_copy(dst_hbm.at[w * NBLK + blk], idx_d)
        gd = [None] * BCH
        gd[0] = pltpu.async_copy(g_hbm.at[idx_s.at[0]], rows.at[0], gsems[0])
        for j in range(BCH):
            b = j % 2
            if j + 1 < BCH:
                gd[j + 1] = pltpu.async_copy(
                    g_hbm.at[idx_s.at[j + 1]], rows.at[1 - b], gsems[1 - b]
                )
            gd[j].wait()
            pltpu.sync_copy(rows.at[b], acc.at[idx_d.at[j]], add=True)
        return _

    lax.fori_loop(0, NBLK, _block, 0)
    plsc.subcore_barrier()
    pltpu.sync_copy(acc.at[pl.ds(s * RPS, RPS)], out_hbm.at[c, pl.ds(s * RPS, RPS)])


BR = 2000


def _tc1_body(degp_ref, x_ref, w_ref, dis_ref, g_ref):
    deg = degp_ref[0, :, 0:1] + degp_ref[1, :, 0:1] + 1.0
    dis = lax.rsqrt(deg)
    dis_ref[...] = jnp.broadcast_to(dis, (BR, DEGW))
    g_ref[...] = jnp.dot(x_ref[...], w_ref[...], preferred_element_type=jnp.float32) * dis


def _tc2_body(sp_ref, g_ref, dis_ref, w_ref, b_ref, g2_ref):
    dis = dis_ref[:, 0:1]
    h = (sp_ref[0] + sp_ref[1] + g_ref[...]) * dis + b_ref[...]
    h = jnp.maximum(h, 0.0)
    g2_ref[...] = jnp.dot(h, w_ref[...], preferred_element_type=jnp.float32) * dis


def _tc3_body(sp_ref, g_ref, dis_ref, b_ref, out_ref):
    dis = dis_ref[:, 0:1]
    h = (sp_ref[0] + sp_ref[1] + g_ref[...]) * dis + b_ref[...]
    m = jnp.max(h, axis=1, keepdims=True)
    ex = jnp.exp(h - m)
    out_ref[...] = h - m - jnp.log(jnp.sum(ex, axis=1, keepdims=True))


_tc1 = pl.pallas_call(
    _tc1_body,
    grid=(N // BR,),
    in_specs=[
        pl.BlockSpec((NC, BR, DEGW), lambda i: (0, i, 0)),
        pl.BlockSpec((BR, D), lambda i: (i, 0)),
        pl.BlockSpec((D, D), lambda i: (0, 0)),
    ],
    out_specs=[
        pl.BlockSpec((BR, DEGW), lambda i: (i, 0)),
        pl.BlockSpec((BR, D), lambda i: (i, 0)),
    ],
    out_shape=[
        jax.ShapeDtypeStruct((N, DEGW), jnp.float32),
        jax.ShapeDtypeStruct((N, D), jnp.float32),
    ],
)

_tc2 = pl.pallas_call(
    _tc2_body,
    grid=(N // BR,),
    in_specs=[
        pl.BlockSpec((NC, BR, D), lambda i: (0, i, 0)),
        pl.BlockSpec((BR, D), lambda i: (i, 0)),
        pl.BlockSpec((BR, DEGW), lambda i: (i, 0)),
        pl.BlockSpec((D, D), lambda i: (0, 0)),
        pl.BlockSpec((1, D), lambda i: (0, 0)),
    ],
    out_specs=pl.BlockSpec((BR, D), lambda i: (i, 0)),
    out_shape=jax.ShapeDtypeStruct((N, D), jnp.float32),
)

_tc3 = pl.pallas_call(
    _tc3_body,
    grid=(N // BR,),
    in_specs=[
        pl.BlockSpec((NC, BR, D), lambda i: (0, i, 0)),
        pl.BlockSpec((BR, D), lambda i: (i, 0)),
        pl.BlockSpec((BR, DEGW), lambda i: (i, 0)),
        pl.BlockSpec((1, D), lambda i: (0, 0)),
    ],
    out_specs=pl.BlockSpec((BR, D), lambda i: (i, 0)),
    out_shape=jax.ShapeDtypeStruct((N, D), jnp.float32),
)


@jax.jit
def kernel(x, edge_index, W1, b1, W2, b2):
    src = edge_index[0].reshape(NW * NBLK, BCH, CH)
    dst = edge_index[1].reshape(NW * NBLK, BCH, CH)
    degp = _sc_deg(edge_index[1])
    dis16, g1 = _tc1(degp, x, W1)
    s1 = _sc_agg(g1, src, dst)
    g2 = _tc2(s1, g1, dis16, W2, b1.reshape(1, D))
    s2 = _sc_agg(g2, src, dst)
    return _tc3(s2, g2, dis16, b2.reshape(1, D))


# speedup vs baseline: 28.6892x; 1.1200x over previous
import functools

import jax
import jax.numpy as jnp
from jax import lax
from jax.experimental import pallas as pl
from jax.experimental.pallas import tpu as pltpu
from jax.experimental.pallas import tpu_sc as plsc

N = 10000
D = 128
E = 320000

NC = 2
NS = 16
NW = NC * NS
EPW = E // NW
CH = 80
NCH = EPW // CH
NP = 10240
RPS = NP // NS
ZR = 128
DEGW = 16
NB = 5
NG = NCH // NB

_MESH = plsc.VectorSubcoreMesh(
    core_axis_name="c", subcore_axis_name="s", num_cores=NC, num_subcores=NS
)


@functools.partial(
    pl.kernel,
    out_type=jax.ShapeDtypeStruct((NC, NP, DEGW), jnp.float32),
    mesh=_MESH,
    scratch_types=[
        pltpu.VMEM_SHARED((NP, DEGW), jnp.float32),
        pltpu.VMEM((CH, DEGW), jnp.float32),
        pltpu.VMEM((CH,), jnp.int32),
        pltpu.VMEM((RPS, DEGW), jnp.float32),
    ],
)
def _sc_deg(dst_hbm, degp_hbm, acc, ones_v, idx_d, zbuf):
    c = lax.axis_index("c")
    s = lax.axis_index("s")
    w = c * NS + s

    zeros16 = jnp.zeros((16,), jnp.float32)
    ones16 = jnp.ones((16,), jnp.float32)

    def _fill(i, _):
        zbuf[i] = zeros16
        return _

    lax.fori_loop(0, RPS, _fill, 0)

    def _fill1(i, _):
        ones_v[i] = ones16
        return _

    lax.fori_loop(0, CH, _fill1, 0)
    pltpu.sync_copy(zbuf, acc.at[pl.ds(s * RPS, RPS)])
    plsc.subcore_barrier()

    base = w * EPW

    def _chunk(i, _):
        pltpu.sync_copy(dst_hbm.at[pl.ds(base + i * CH, CH)], idx_d)
        pltpu.sync_copy(ones_v, acc.at[idx_d], add=True)
        return _

    lax.fori_loop(0, NCH, _chunk, 0)
    plsc.subcore_barrier()
    pltpu.sync_copy(acc.at[pl.ds(s * RPS, RPS)], degp_hbm.at[c, pl.ds(s * RPS, RPS)])


NBUF = 4
BCH = 25
NBLK = NCH // BCH


@functools.partial(
    pl.kernel,
    out_type=jax.ShapeDtypeStruct((NC, NP, D), jnp.float32),
    mesh=_MESH,
    scratch_types=[
        pltpu.VMEM_SHARED((NP, D), jnp.float32),
        pltpu.VMEM((BCH, CH), jnp.int32),
        pltpu.VMEM((BCH, CH), jnp.int32),
        pltpu.VMEM((NBUF, CH, D), jnp.float32),
        [pltpu.SemaphoreType.DMA] * NBUF,
        [pltpu.SemaphoreType.DMA] * NBUF,
    ],
)
def _sc_agg(g_hbm, src_hbm, dst_hbm, out_hbm, acc, idx_s, idx_d, rows, gsems, ssems):
    c = lax.axis_index("c")
    s = lax.axis_index("s")
    w = c * NS + s

    zeros16 = jnp.zeros((16,), jnp.float32)

    def _fill(i, _):
        for b in range(NBUF):
            for j in range(D // 16):
                rows[b, i, pl.ds(j * 16, 16)] = zeros16
        return _

    lax.fori_loop(0, CH, _fill, 0)

    def _zero(k, _):
        for b in range(NBUF):
            pltpu.sync_copy(
                rows.at[b], acc.at[pl.ds(s * RPS + (k * NBUF + b) * CH, CH)]
            )
        return _

    lax.fori_loop(0, RPS // (NBUF * CH), _zero, 0)
    plsc.subcore_barrier()

    def _block(blk, _):
        pltpu.sync_copy(src_hbm.at[w * NBLK + blk], idx_s)
        pltpu.sync_copy(dst_hbm.at[w * NBLK + blk], idx_d)
        gd = [None] * BCH
        sd = [None] * BCH
        for j in range(BCH):
            b = j % NBUF
            if j >= NBUF:
                sd[j - NBUF].wait()
            gd[j] = pltpu.async_copy(g_hbm.at[idx_s.at[j]], rows.at[b], gsems[b])
            if j >= NBUF - 1:
                k = j - (NBUF - 1)
                gd[k].wait()
                sd[k] = pltpu.async_copy(
                    rows.at[k % NBUF], acc.at[idx_d.at[k]], ssems[k % NBUF], add=True
                )
        for k in range(BCH - (NBUF - 1), BCH):
            gd[k].wait()
            sd[k] = pltpu.async_copy(
                rows.at[k % NBUF], acc.at[idx_d.at[k]], ssems[k % NBUF], add=True
            )
        for k in range(BCH - NBUF, BCH):
            sd[k].wait()
        return _

    lax.fori_loop(0, NBLK, _block, 0)
    plsc.subcore_barrier()
    pltpu.sync_copy(acc.at[pl.ds(s * RPS, RPS)], out_hbm.at[c, pl.ds(s * RPS, RPS)])


BR = 2000


def _tc1_body(degp_ref, x_ref, w_ref, dis_ref, g_ref):
    deg = degp_ref[0, :, 0:1] + degp_ref[1, :, 0:1] + 1.0
    dis = lax.rsqrt(deg)
    dis_ref[...] = jnp.broadcast_to(dis, (BR, DEGW))
    g_ref[...] = jnp.dot(x_ref[...], w_ref[...], preferred_element_type=jnp.float32) * dis


def _tc2_body(sp_ref, g_ref, dis_ref, w_ref, b_ref, g2_ref):
    dis = dis_ref[:, 0:1]
    h = (sp_ref[0] + sp_ref[1] + g_ref[...]) * dis + b_ref[...]
    h = jnp.maximum(h, 0.0)
    g2_ref[...] = jnp.dot(h, w_ref[...], preferred_element_type=jnp.float32) * dis


def _tc3_body(sp_ref, g_ref, dis_ref, b_ref, out_ref):
    dis = dis_ref[:, 0:1]
    h = (sp_ref[0] + sp_ref[1] + g_ref[...]) * dis + b_ref[...]
    m = jnp.max(h, axis=1, keepdims=True)
    ex = jnp.exp(h - m)
    out_ref[...] = h - m - jnp.log(jnp.sum(ex, axis=1, keepdims=True))


_tc1 = pl.pallas_call(
    _tc1_body,
    grid=(N // BR,),
    in_specs=[
        pl.BlockSpec((NC, BR, DEGW), lambda i: (0, i, 0)),
        pl.BlockSpec((BR, D), lambda i: (i, 0)),
        pl.BlockSpec((D, D), lambda i: (0, 0)),
    ],
    out_specs=[
        pl.BlockSpec((BR, DEGW), lambda i: (i, 0)),
        pl.BlockSpec((BR, D), lambda i: (i, 0)),
    ],
    out_shape=[
        jax.ShapeDtypeStruct((N, DEGW), jnp.float32),
        jax.ShapeDtypeStruct((N, D), jnp.float32),
    ],
)

_tc2 = pl.pallas_call(
    _tc2_body,
    grid=(N // BR,),
    in_specs=[
        pl.BlockSpec((NC, BR, D), lambda i: (0, i, 0)),
        pl.BlockSpec((BR, D), lambda i: (i, 0)),
        pl.BlockSpec((BR, DEGW), lambda i: (i, 0)),
        pl.BlockSpec((D, D), lambda i: (0, 0)),
        pl.BlockSpec((1, D), lambda i: (0, 0)),
    ],
    out_specs=pl.BlockSpec((BR, D), lambda i: (i, 0)),
    out_shape=jax.ShapeDtypeStruct((N, D), jnp.float32),
)

_tc3 = pl.pallas_call(
    _tc3_body,
    grid=(N // BR,),
    in_specs=[
        pl.BlockSpec((NC, BR, D), lambda i: (0, i, 0)),
        pl.BlockSpec((BR, D), lambda i: (i, 0)),
        pl.BlockSpec((BR, DEGW), lambda i: (i, 0)),
        pl.BlockSpec((1, D), lambda i: (0, 0)),
    ],
    out_specs=pl.BlockSpec((BR, D), lambda i: (i, 0)),
    out_shape=jax.ShapeDtypeStruct((N, D), jnp.float32),
)


@jax.jit
def kernel(x, edge_index, W1, b1, W2, b2):
    src = edge_index[0].reshape(NW * NBLK, BCH, CH)
    dst = edge_index[1].reshape(NW * NBLK, BCH, CH)
    degp = _sc_deg(edge_index[1])
    dis16, g1 = _tc1(degp, x, W1)
    s1 = _sc_agg(g1, src, dst)
    g2 = _tc2(s1, g1, dis16, W2, b1.reshape(1, D))
    s2 = _sc_agg(g2, src, dst)
    return _tc3(s2, g2, dis16, b2.reshape(1, D))


# speedup vs baseline: 34.0104x; 1.1855x over previous
import functools

import jax
import jax.numpy as jnp
from jax import lax
from jax.experimental import pallas as pl
from jax.experimental.pallas import tpu as pltpu
from jax.experimental.pallas import tpu_sc as plsc

N = 10000
D = 128
E = 320000

NC = 2
NS = 16
NW = NC * NS
EPW = E // NW
CH = 80
NCH = EPW // CH
NP = 10240
RPS = NP // NS
ZR = 128
DEGW = 16
NB = 5
NG = NCH // NB

_MESH = plsc.VectorSubcoreMesh(
    core_axis_name="c", subcore_axis_name="s", num_cores=NC, num_subcores=NS
)


@functools.partial(
    pl.kernel,
    out_type=jax.ShapeDtypeStruct((NC, NP, DEGW), jnp.float32),
    mesh=_MESH,
    scratch_types=[
        pltpu.VMEM_SHARED((NP, DEGW), jnp.float32),
        pltpu.VMEM((CH, DEGW), jnp.float32),
        pltpu.VMEM((25, CH), jnp.int32),
        pltpu.VMEM((RPS, DEGW), jnp.float32),
        [pltpu.SemaphoreType.DMA] * 4,
    ],
)
def _sc_deg(dst_hbm, degp_hbm, acc, ones_v, idx_d, zbuf, ssems):
    c = lax.axis_index("c")
    s = lax.axis_index("s")
    w = c * NS + s

    zeros16 = jnp.zeros((16,), jnp.float32)
    ones16 = jnp.ones((16,), jnp.float32)

    def _fill(i, _):
        zbuf[i] = zeros16
        return _

    lax.fori_loop(0, RPS, _fill, 0)

    def _fill1(i, _):
        ones_v[i] = ones16
        return _

    lax.fori_loop(0, CH, _fill1, 0)
    pltpu.sync_copy(zbuf, acc.at[pl.ds(s * RPS, RPS)])
    plsc.subcore_barrier()

    def _block(blk, _):
        pltpu.sync_copy(dst_hbm.at[w * 5 + blk], idx_d)
        sd = [None] * 25
        for j in range(25):
            if j >= 4:
                sd[j - 4].wait()
            sd[j] = pltpu.async_copy(ones_v, acc.at[idx_d.at[j]], ssems[j % 4], add=True)
        for k in range(21, 25):
            sd[k].wait()
        return _

    lax.fori_loop(0, 5, _block, 0)
    plsc.subcore_barrier()
    pltpu.sync_copy(acc.at[pl.ds(s * RPS, RPS)], degp_hbm.at[c, pl.ds(s * RPS, RPS)])


NBUF = 4
BCH = 25
NBLK = NCH // BCH


@functools.partial(
    pl.kernel,
    out_type=jax.ShapeDtypeStruct((NC, NP, D), jnp.float32),
    mesh=_MESH,
    scratch_types=[
        pltpu.VMEM_SHARED((NP, D), jnp.float32),
        pltpu.VMEM((BCH, CH), jnp.int32),
        pltpu.VMEM((BCH, CH), jnp.int32),
        pltpu.VMEM((NBUF, CH, D), jnp.float32),
        [pltpu.SemaphoreType.DMA] * NBUF,
        [pltpu.SemaphoreType.DMA] * NBUF,
    ],
)
def _sc_agg(g_hbm, src_hbm, dst_hbm, out_hbm, acc, idx_s, idx_d, rows, gsems, ssems):
    c = lax.axis_index("c")
    s = lax.axis_index("s")
    w = c * NS + s

    zeros16 = jnp.zeros((16,), jnp.float32)

    def _fill(i, _):
        for b in range(NBUF):
            for j in range(D // 16):
                rows[b, i, pl.ds(j * 16, 16)] = zeros16
        return _

    lax.fori_loop(0, CH, _fill, 0)

    def _zero(k, _):
        for b in range(NBUF):
            pltpu.sync_copy(
                rows.at[b], acc.at[pl.ds(s * RPS + (k * NBUF + b) * CH, CH)]
            )
        return _

    lax.fori_loop(0, RPS // (NBUF * CH), _zero, 0)
    plsc.subcore_barrier()

    def _block(blk, _):
        pltpu.sync_copy(src_hbm.at[w * NBLK + blk], idx_s)
        pltpu.sync_copy(dst_hbm.at[w * NBLK + blk], idx_d)
        gd = [None] * BCH
        sd = [None] * BCH
        for j in range(BCH):
            b = j % NBUF
            if j >= NBUF:
                sd[j - NBUF].wait()
            gd[j] = pltpu.async_copy(g_hbm.at[idx_s.at[j]], rows.at[b], gsems[b])
            if j >= NBUF - 1:
                k = j - (NBUF - 1)
                gd[k].wait()
                sd[k] = pltpu.async_copy(
                    rows.at[k % NBUF], acc.at[idx_d.at[k]], ssems[k % NBUF], add=True
                )
        for k in range(BCH - (NBUF - 1), BCH):
            gd[k].wait()
            sd[k] = pltpu.async_copy(
                rows.at[k % NBUF], acc.at[idx_d.at[k]], ssems[k % NBUF], add=True
            )
        for k in range(BCH - NBUF, BCH):
            sd[k].wait()
        return _

    lax.fori_loop(0, NBLK, _block, 0)
    plsc.subcore_barrier()
    pltpu.sync_copy(acc.at[pl.ds(s * RPS, RPS)], out_hbm.at[c, pl.ds(s * RPS, RPS)])


BR = 2000


def _tc1_body(degp_ref, x_ref, w_ref, dis_ref, g_ref):
    deg = degp_ref[0, :, 0:1] + degp_ref[1, :, 0:1] + 1.0
    dis = lax.rsqrt(deg)
    dis_ref[...] = jnp.broadcast_to(dis, (BR, DEGW))
    g_ref[...] = jnp.dot(x_ref[...], w_ref[...], preferred_element_type=jnp.float32) * dis


def _tc2_body(sp_ref, g_ref, dis_ref, w_ref, b_ref, g2_ref):
    dis = dis_ref[:, 0:1]
    h = (sp_ref[0] + sp_ref[1] + g_ref[...]) * dis + b_ref[...]
    h = jnp.maximum(h, 0.0)
    g2_ref[...] = jnp.dot(h, w_ref[...], preferred_element_type=jnp.float32) * dis


def _tc3_body(sp_ref, g_ref, dis_ref, b_ref, out_ref):
    dis = dis_ref[:, 0:1]
    h = (sp_ref[0] + sp_ref[1] + g_ref[...]) * dis + b_ref[...]
    m = jnp.max(h, axis=1, keepdims=True)
    ex = jnp.exp(h - m)
    out_ref[...] = h - m - jnp.log(jnp.sum(ex, axis=1, keepdims=True))


_tc1 = pl.pallas_call(
    _tc1_body,
    grid=(N // BR,),
    in_specs=[
        pl.BlockSpec((NC, BR, DEGW), lambda i: (0, i, 0)),
        pl.BlockSpec((BR, D), lambda i: (i, 0)),
        pl.BlockSpec((D, D), lambda i: (0, 0)),
    ],
    out_specs=[
        pl.BlockSpec((BR, DEGW), lambda i: (i, 0)),
        pl.BlockSpec((BR, D), lambda i: (i, 0)),
    ],
    out_shape=[
        jax.ShapeDtypeStruct((N, DEGW), jnp.float32),
        jax.ShapeDtypeStruct((N, D), jnp.float32),
    ],
)

_tc2 = pl.pallas_call(
    _tc2_body,
    grid=(N // BR,),
    in_specs=[
        pl.BlockSpec((NC, BR, D), lambda i: (0, i, 0)),
        pl.BlockSpec((BR, D), lambda i: (i, 0)),
        pl.BlockSpec((BR, DEGW), lambda i: (i, 0)),
        pl.BlockSpec((D, D), lambda i: (0, 0)),
        pl.BlockSpec((1, D), lambda i: (0, 0)),
    ],
    out_specs=pl.BlockSpec((BR, D), lambda i: (i, 0)),
    out_shape=jax.ShapeDtypeStruct((N, D), jnp.float32),
)

_tc3 = pl.pallas_call(
    _tc3_body,
    grid=(N // BR,),
    in_specs=[
        pl.BlockSpec((NC, BR, D), lambda i: (0, i, 0)),
        pl.BlockSpec((BR, D), lambda i: (i, 0)),
        pl.BlockSpec((BR, DEGW), lambda i: (i, 0)),
        pl.BlockSpec((1, D), lambda i: (0, 0)),
    ],
    out_specs=pl.BlockSpec((BR, D), lambda i: (i, 0)),
    out_shape=jax.ShapeDtypeStruct((N, D), jnp.float32),
)


@jax.jit
def kernel(x, edge_index, W1, b1, W2, b2):
    src = edge_index[0].reshape(NW * NBLK, BCH, CH)
    dst = edge_index[1].reshape(NW * NBLK, BCH, CH)
    degp = _sc_deg(dst)
    dis16, g1 = _tc1(degp, x, W1)
    s1 = _sc_agg(g1, src, dst)
    g2 = _tc2(s1, g1, dis16, W2, b1.reshape(1, D))
    s2 = _sc_agg(g2, src, dst)
    return _tc3(s2, g2, dis16, b2.reshape(1, D))


# speedup vs baseline: 34.0265x; 1.0005x over previous
import functools

import jax
import jax.numpy as jnp
from jax import lax
from jax.experimental import pallas as pl
from jax.experimental.pallas import tpu as pltpu
from jax.experimental.pallas import tpu_sc as plsc

N = 10000
D = 128
E = 320000

NC = 2
NS = 16
NW = NC * NS
EPW = E // NW
CH = 80
NCH = EPW // CH
NP = 10240
RPS = NP // NS
ZR = 128
DEGW = 16
NBUF = 4
BCH = 25
NBLK = NCH // BCH

_MESH = plsc.VectorSubcoreMesh(
    core_axis_name="c", subcore_axis_name="s", num_cores=NC, num_subcores=NS
)


@functools.partial(
    pl.kernel,
    out_type=jax.ShapeDtypeStruct((NC, NP, DEGW), jnp.float32),
    mesh=_MESH,
    scratch_types=[
        pltpu.VMEM_SHARED((NP, DEGW), jnp.float32),
        pltpu.VMEM((CH, DEGW), jnp.float32),
        [pltpu.VMEM((CH,), jnp.int32)] * NBUF,
        pltpu.VMEM((RPS, DEGW), jnp.float32),
        [pltpu.SemaphoreType.DMA] * NBUF,
        [pltpu.SemaphoreType.DMA] * NBUF,
    ],
)
def _sc_deg(dst_hbm, degp_hbm, acc, ones_v, idx_slots, zbuf, dsems, ssems):
    c = lax.axis_index("c")
    s = lax.axis_index("s")
    w = c * NS + s

    zeros16 = jnp.zeros((16,), jnp.float32)
    ones16 = jnp.ones((16,), jnp.float32)

    def _fill(i, _):
        zbuf[i] = zeros16
        return _

    lax.fori_loop(0, RPS, _fill, 0)

    def _fill1(i, _):
        ones_v[i] = ones16
        return _

    lax.fori_loop(0, CH, _fill1, 0)
    pltpu.sync_copy(zbuf, acc.at[pl.ds(s * RPS, RPS)])
    plsc.subcore_barrier()

    base = w * EPW

    def _block(blk, _):
        b0 = base + blk * BCH * CH
        dd = [None] * BCH
        sd = [None] * BCH
        for j in range(BCH):
            b = j % NBUF
            if j >= NBUF:
                sd[j - NBUF].wait()
            dd[j] = pltpu.async_copy(
                dst_hbm.at[pl.ds(b0 + j * CH, CH)], idx_slots[b], dsems[b]
            )
            if j >= NBUF - 1:
                k = j - (NBUF - 1)
                dd[k].wait()
                sd[k] = pltpu.async_copy(
                    ones_v, acc.at[idx_slots[k % NBUF]], ssems[k % NBUF], add=True
                )
        for k in range(BCH - (NBUF - 1), BCH):
            dd[k].wait()
            sd[k] = pltpu.async_copy(
                ones_v, acc.at[idx_slots[k % NBUF]], ssems[k % NBUF], add=True
            )
        for k in range(BCH - NBUF, BCH):
            sd[k].wait()
        return _

    lax.fori_loop(0, NBLK, _block, 0)
    plsc.subcore_barrier()
    pltpu.sync_copy(acc.at[pl.ds(s * RPS, RPS)], degp_hbm.at[c, pl.ds(s * RPS, RPS)])


@functools.partial(
    pl.kernel,
    out_type=jax.ShapeDtypeStruct((NC, NP, D), jnp.float32),
    mesh=_MESH,
    scratch_types=[
        pltpu.VMEM_SHARED((NP, D), jnp.float32),
        pltpu.VMEM((BCH * CH,), jnp.int32),
        [pltpu.VMEM((CH,), jnp.int32)] * NBUF,
        pltpu.VMEM((NBUF, CH, D), jnp.float32),
        [pltpu.SemaphoreType.DMA] * NBUF,
        [pltpu.SemaphoreType.DMA] * NBUF,
        [pltpu.SemaphoreType.DMA] * NBUF,
    ],
)
def _sc_agg(g_hbm, src_hbm, dst_hbm, out_hbm, acc, idx_s, idx_slots, rows, gsems, dsems, ssems):
    c = lax.axis_index("c")
    s = lax.axis_index("s")
    w = c * NS + s

    zeros16 = jnp.zeros((16,), jnp.float32)

    def _fill(i, _):
        for b in range(NBUF):
            for j in range(D // 16):
                rows[b, i, pl.ds(j * 16, 16)] = zeros16
        return _

    lax.fori_loop(0, CH, _fill, 0)

    def _zero(k, _):
        for b in range(NBUF):
            pltpu.sync_copy(
                rows.at[b], acc.at[pl.ds(s * RPS + (k * NBUF + b) * CH, CH)]
            )
        return _

    lax.fori_loop(0, RPS // (NBUF * CH), _zero, 0)
    plsc.subcore_barrier()

    base = w * EPW

    def _block(blk, _):
        b0 = base + blk * BCH * CH
        pltpu.sync_copy(src_hbm.at[pl.ds(b0, BCH * CH)], idx_s)
        gd = [None] * BCH
        dd = [None] * BCH
        sd = [None] * BCH
        for j in range(BCH):
            b = j % NBUF
            if j >= NBUF:
                sd[j - NBUF].wait()
            dd[j] = pltpu.async_copy(
                dst_hbm.at[pl.ds(b0 + j * CH, CH)], idx_slots[b], dsems[b]
            )
            gd[j] = pltpu.async_copy(
                g_hbm.at[idx_s.at[pl.ds(j * CH, CH)]], rows.at[b], gsems[b]
            )
            if j >= NBUF - 1:
                k = j - (NBUF - 1)
                gd[k].wait()
                dd[k].wait()
                sd[k] = pltpu.async_copy(
                    rows.at[k % NBUF], acc.at[idx_slots[k % NBUF]], ssems[k % NBUF], add=True
                )
        for k in range(BCH - (NBUF - 1), BCH):
            gd[k].wait()
            dd[k].wait()
            sd[k] = pltpu.async_copy(
                rows.at[k % NBUF], acc.at[idx_slots[k % NBUF]], ssems[k % NBUF], add=True
            )
        for k in range(BCH - NBUF, BCH):
            sd[k].wait()
        return _

    lax.fori_loop(0, NBLK, _block, 0)
    plsc.subcore_barrier()
    pltpu.sync_copy(acc.at[pl.ds(s * RPS, RPS)], out_hbm.at[c, pl.ds(s * RPS, RPS)])


BR = 2000


def _tc1_body(degp_ref, x_ref, w_ref, dis_ref, g_ref):
    deg = degp_ref[0, :, 0:1] + degp_ref[1, :, 0:1] + 1.0
    dis = lax.rsqrt(deg)
    dis_ref[...] = jnp.broadcast_to(dis, (BR, DEGW))
    g_ref[...] = jnp.dot(x_ref[...], w_ref[...], preferred_element_type=jnp.float32) * dis


def _tc2_body(sp_ref, g_ref, dis_ref, w_ref, b_ref, g2_ref):
    dis = dis_ref[:, 0:1]
    h = (sp_ref[0] + sp_ref[1] + g_ref[...]) * dis + b_ref[...]
    h = jnp.maximum(h, 0.0)
    g2_ref[...] = jnp.dot(h, w_ref[...], preferred_element_type=jnp.float32) * dis


def _tc3_body(sp_ref, g_ref, dis_ref, b_ref, out_ref):
    dis = dis_ref[:, 0:1]
    h = (sp_ref[0] + sp_ref[1] + g_ref[...]) * dis + b_ref[...]
    m = jnp.max(h, axis=1, keepdims=True)
    ex = jnp.exp(h - m)
    out_ref[...] = h - m - jnp.log(jnp.sum(ex, axis=1, keepdims=True))


_tc1 = pl.pallas_call(
    _tc1_body,
    grid=(N // BR,),
    in_specs=[
        pl.BlockSpec((NC, BR, DEGW), lambda i: (0, i, 0)),
        pl.BlockSpec((BR, D), lambda i: (i, 0)),
        pl.BlockSpec((D, D), lambda i: (0, 0)),
    ],
    out_specs=[
        pl.BlockSpec((BR, DEGW), lambda i: (i, 0)),
        pl.BlockSpec((BR, D), lambda i: (i, 0)),
    ],
    out_shape=[
        jax.ShapeDtypeStruct((N, DEGW), jnp.float32),
        jax.ShapeDtypeStruct((N, D), jnp.float32),
    ],
)

_tc2 = pl.pallas_call(
    _tc2_body,
    grid=(N // BR,),
    in_specs=[
        pl.BlockSpec((NC, BR, D), lambda i: (0, i, 0)),
        pl.BlockSpec((BR, D), lambda i: (i, 0)),
        pl.BlockSpec((BR, DEGW), lambda i: (i, 0)),
        pl.BlockSpec((D, D), lambda i: (0, 0)),
        pl.BlockSpec((1, D), lambda i: (0, 0)),
    ],
    out_specs=pl.BlockSpec((BR, D), lambda i: (i, 0)),
    out_shape=jax.ShapeDtypeStruct((N, D), jnp.float32),
)

_tc3 = pl.pallas_call(
    _tc3_body,
    grid=(N // BR,),
    in_specs=[
        pl.BlockSpec((NC, BR, D), lambda i: (0, i, 0)),
        pl.BlockSpec((BR, D), lambda i: (i, 0)),
        pl.BlockSpec((BR, DEGW), lambda i: (i, 0)),
        pl.BlockSpec((1, D), lambda i: (0, 0)),
    ],
    out_specs=pl.BlockSpec((BR, D), lambda i: (i, 0)),
    out_shape=jax.ShapeDtypeStruct((N, D), jnp.float32),
)


@jax.jit
def kernel(x, edge_index, W1, b1, W2, b2):
    src = edge_index[0]
    dst = edge_index[1]
    degp = _sc_deg(dst)
    dis16, g1 = _tc1(degp, x, W1)
    s1 = _sc_agg(g1, src, dst)
    g2 = _tc2(s1, g1, dis16, W2, b1.reshape(1, D))
    s2 = _sc_agg(g2, src, dst)
    return _tc3(s2, g2, dis16, b2.reshape(1, D))


# speedup vs baseline: 34.2513x; 1.0066x over previous
import functools

import jax
import jax.numpy as jnp
from jax import lax
from jax.experimental import pallas as pl
from jax.experimental.pallas import tpu as pltpu
from jax.experimental.pallas import tpu_sc as plsc

N = 10000
D = 128
E = 320000

NC = 2
NS = 16
NW = NC * NS
EPW = E // NW
CH = 80
NCH = EPW // CH
NP = 10240
RPS = NP // NS
ZR = 128
DEGW = 16
NBUF = 4
BCH = 25
NBLK = NCH // BCH

_MESH = plsc.VectorSubcoreMesh(
    core_axis_name="c", subcore_axis_name="s", num_cores=NC, num_subcores=NS
)


@functools.partial(
    pl.kernel,
    out_type=jax.ShapeDtypeStruct((NC, NP, DEGW), jnp.float32),
    mesh=_MESH,
    scratch_types=[
        pltpu.VMEM_SHARED((NP, DEGW), jnp.float32),
        pltpu.VMEM((CH, DEGW), jnp.float32),
        pltpu.VMEM((BCH * CH,), jnp.int32),
        pltpu.VMEM((RPS, DEGW), jnp.float32),
        [pltpu.SemaphoreType.DMA] * NBUF,
    ],
)
def _sc_deg(dst_hbm, degp_hbm, acc, ones_v, idx_d, zbuf, ssems):
    c = lax.axis_index("c")
    s = lax.axis_index("s")
    w = c * NS + s

    zeros16 = jnp.zeros((16,), jnp.float32)
    ones16 = jnp.ones((16,), jnp.float32)

    def _fill(i, _):
        zbuf[i] = zeros16
        return _

    lax.fori_loop(0, RPS, _fill, 0)

    def _fill1(i, _):
        ones_v[i] = ones16
        return _

    lax.fori_loop(0, CH, _fill1, 0)
    pltpu.sync_copy(zbuf, acc.at[pl.ds(s * RPS, RPS)])
    plsc.subcore_barrier()

    base = w * EPW

    def _block(blk, _):
        b0 = base + blk * BCH * CH
        pltpu.sync_copy(dst_hbm.at[pl.ds(b0, BCH * CH)], idx_d)
        sd = [None] * BCH
        for j in range(BCH):
            if j >= NBUF:
                sd[j - NBUF].wait()
            sd[j] = pltpu.async_copy(
                ones_v, acc.at[idx_d.at[pl.ds(j * CH, CH)]], ssems[j % NBUF], add=True
            )
        for k in range(BCH - NBUF, BCH):
            sd[k].wait()
        return _

    lax.fori_loop(0, NBLK, _block, 0)
    plsc.subcore_barrier()
    pltpu.sync_copy(acc.at[pl.ds(s * RPS, RPS)], degp_hbm.at[c, pl.ds(s * RPS, RPS)])


@functools.partial(
    pl.kernel,
    out_type=jax.ShapeDtypeStruct((NC, NP, D), jnp.float32),
    mesh=_MESH,
    scratch_types=[
        pltpu.VMEM_SHARED((NP, D), jnp.float32),
        pltpu.VMEM((BCH * CH,), jnp.int32),
        pltpu.VMEM((BCH * CH,), jnp.int32),
        pltpu.VMEM((NBUF, CH, D), jnp.float32),
        [pltpu.SemaphoreType.DMA] * NBUF,
        [pltpu.SemaphoreType.DMA] * NBUF,
    ],
)
def _sc_agg(g_hbm, src_hbm, dst_hbm, out_hbm, acc, idx_s, idx_d, rows, gsems, ssems):
    c = lax.axis_index("c")
    s = lax.axis_index("s")
    w = c * NS + s

    zeros16 = jnp.zeros((16,), jnp.float32)

    def _fill(i, _):
        for b in range(NBUF):
            for j in range(D // 16):
                rows[b, i, pl.ds(j * 16, 16)] = zeros16
        return _

    lax.fori_loop(0, CH, _fill, 0)

    def _zero(k, _):
        for b in range(NBUF):
            pltpu.sync_copy(
                rows.at[b], acc.at[pl.ds(s * RPS + (k * NBUF + b) * CH, CH)]
            )
        return _

    lax.fori_loop(0, RPS // (NBUF * CH), _zero, 0)
    plsc.subcore_barrier()

    base = w * EPW

    def _block(blk, _):
        b0 = base + blk * BCH * CH
        pltpu.sync_copy(src_hbm.at[pl.ds(b0, BCH * CH)], idx_s)
        pltpu.sync_copy(dst_hbm.at[pl.ds(b0, BCH * CH)], idx_d)
        gd = [None] * BCH
        sd = [None] * BCH
        for j in range(BCH):
            b = j % NBUF
            if j >= NBUF:
                sd[j - NBUF].wait()
            gd[j] = pltpu.async_copy(
                g_hbm.at[idx_s.at[pl.ds(j * CH, CH)]], rows.at[b], gsems[b]
            )
            if j >= NBUF - 1:
                k = j - (NBUF - 1)
                gd[k].wait()
                sd[k] = pltpu.async_copy(
                    rows.at[k % NBUF], acc.at[idx_d.at[pl.ds(k * CH, CH)]], ssems[k % NBUF], add=True
                )
        for k in range(BCH - (NBUF - 1), BCH):
            gd[k].wait()
            sd[k] = pltpu.async_copy(
                rows.at[k % NBUF], acc.at[idx_d.at[pl.ds(k * CH, CH)]], ssems[k % NBUF], add=True
            )
        for k in range(BCH - NBUF, BCH):
            sd[k].wait()
        return _

    lax.fori_loop(0, NBLK, _block, 0)
    plsc.subcore_barrier()
    pltpu.sync_copy(acc.at[pl.ds(s * RPS, RPS)], out_hbm.at[c, pl.ds(s * RPS, RPS)])


BR = 2000


def _tc1_body(degp_ref, x_ref, w_ref, dis_ref, g_ref):
    deg = degp_ref[0, :, 0:1] + degp_ref[1, :, 0:1] + 1.0
    dis = lax.rsqrt(deg)
    dis_ref[...] = jnp.broadcast_to(dis, (BR, DEGW))
    g_ref[...] = jnp.dot(x_ref[...], w_ref[...], preferred_element_type=jnp.float32) * dis


def _tc2_body(sp_ref, g_ref, dis_ref, w_ref, b_ref, g2_ref):
    dis = dis_ref[:, 0:1]
    h = (sp_ref[0] + sp_ref[1] + g_ref[...]) * dis + b_ref[...]
    h = jnp.maximum(h, 0.0)
    g2_ref[...] = jnp.dot(h, w_ref[...], preferred_element_type=jnp.float32) * dis


def _tc3_body(sp_ref, g_ref, dis_ref, b_ref, out_ref):
    dis = dis_ref[:, 0:1]
    h = (sp_ref[0] + sp_ref[1] + g_ref[...]) * dis + b_ref[...]
    m = jnp.max(h, axis=1, keepdims=True)
    ex = jnp.exp(h - m)
    out_ref[...] = h - m - jnp.log(jnp.sum(ex, axis=1, keepdims=True))


_tc1 = pl.pallas_call(
    _tc1_body,
    grid=(N // BR,),
    in_specs=[
        pl.BlockSpec((NC, BR, DEGW), lambda i: (0, i, 0)),
        pl.BlockSpec((BR, D), lambda i: (i, 0)),
        pl.BlockSpec((D, D), lambda i: (0, 0)),
    ],
    out_specs=[
        pl.BlockSpec((BR, DEGW), lambda i: (i, 0)),
        pl.BlockSpec((BR, D), lambda i: (i, 0)),
    ],
    out_shape=[
        jax.ShapeDtypeStruct((N, DEGW), jnp.float32),
        jax.ShapeDtypeStruct((N, D), jnp.float32),
    ],
)

_tc2 = pl.pallas_call(
    _tc2_body,
    grid=(N // BR,),
    in_specs=[
        pl.BlockSpec((NC, BR, D), lambda i: (0, i, 0)),
        pl.BlockSpec((BR, D), lambda i: (i, 0)),
        pl.BlockSpec((BR, DEGW), lambda i: (i, 0)),
        pl.BlockSpec((D, D), lambda i: (0, 0)),
        pl.BlockSpec((1, D), lambda i: (0, 0)),
    ],
    out_specs=pl.BlockSpec((BR, D), lambda i: (i, 0)),
    out_shape=jax.ShapeDtypeStruct((N, D), jnp.float32),
)

_tc3 = pl.pallas_call(
    _tc3_body,
    grid=(N // BR,),
    in_specs=[
        pl.BlockSpec((NC, BR, D), lambda i: (0, i, 0)),
        pl.BlockSpec((BR, D), lambda i: (i, 0)),
        pl.BlockSpec((BR, DEGW), lambda i: (i, 0)),
        pl.BlockSpec((1, D), lambda i: (0, 0)),
    ],
    out_specs=pl.BlockSpec((BR, D), lambda i: (i, 0)),
    out_shape=jax.ShapeDtypeStruct((N, D), jnp.float32),
)


@jax.jit
def kernel(x, edge_index, W1, b1, W2, b2):
    src = edge_index[0]
    dst = edge_index[1]
    degp = _sc_deg(dst)
    dis16, g1 = _tc1(degp, x, W1)
    s1 = _sc_agg(g1, src, dst)
    g2 = _tc2(s1, g1, dis16, W2, b1.reshape(1, D))
    s2 = _sc_agg(g2, src, dst)
    return _tc3(s2, g2, dis16, b2.reshape(1, D))
